# baseline (speedup 1.0000x reference)
import functools

import jax
import jax.numpy as jnp
from jax import lax
from jax.experimental import pallas as pl
from jax.experimental.pallas import tpu as pltpu
from jax.experimental.pallas import tpu_sc as plsc

N_NODES = 10000
N_INC = 320000
D_FEAT = 128
N_HEDGES = 10000

NC = 2
NS = 16
NW = NC * NS
CH = 128
NK = 80
NKH = NK // 2
N_INC_PAD = NW * NK * CH
NPAD = 10112
TRASH_ROW = 10050
ROWS_PER_TILE = NPAD // NS
NBUF = 2

_f32 = jnp.float32
_i32 = jnp.int32

_MESH = plsc.VectorSubcoreMesh(
    core_axis_name="c", subcore_axis_name="s", num_cores=NC, num_subcores=NS)


def _sc_hop_body(gidx_hbm, widx_hbm, table_hbm, zeros_hbm, out_hbm,
                 gi_v, wi_v, rows_v, acc_sh, *sems):
    gsems = sems[:NBUF]
    ssems = sems[NBUF:]
    cid = lax.axis_index("c")
    sid = lax.axis_index("s")
    wid = sid * NC + cid
    strip = pl.ds(sid * ROWS_PER_TILE, ROWS_PER_TILE)

    pltpu.sync_copy(zeros_hbm.at[strip], acc_sh.at[strip])
    plsc.subcore_barrier()

    for h in range(2):
        pltpu.sync_copy(gidx_hbm.at[pl.ds(wid * NK + h * NKH, NKH)], gi_v)
        pltpu.sync_copy(widx_hbm.at[pl.ds(wid * NK + h * NKH, NKH)], wi_v)

        def pair(j, carry):
            gds = []
            for b in range(NBUF):
                gds.append(pltpu.async_copy(
                    table_hbm.at[gi_v.at[j * NBUF + b]], rows_v.at[b],
                    gsems[b]))
            sds = []
            for b in range(NBUF):
                gds[b].wait()
                sds.append(pltpu.async_copy(
                    rows_v.at[b], acc_sh.at[wi_v.at[j * NBUF + b]], ssems[b],
                    add=True))
            for b in range(NBUF):
                sds[b].wait()
            return carry

        lax.fori_loop(0, NKH // NBUF, pair, 0)
    plsc.subcore_barrier()
    pltpu.sync_copy(acc_sh.at[strip], out_hbm.at[cid, strip])


def _make_sc_hop(feat):
    return pl.kernel(
        _sc_hop_body,
        mesh=_MESH,
        out_type=jax.ShapeDtypeStruct((NC, NPAD, feat), _f32),
        compiler_params=pltpu.CompilerParams(use_tc_tiling_on_sc=False),
        scratch_types=[
            pltpu.VMEM((NKH, CH), _i32),
            pltpu.VMEM((NKH, CH), _i32),
            pltpu.VMEM((NBUF, CH, feat), _f32),
            pltpu.VMEM_SHARED((NPAD, feat), _f32),
        ] + [pltpu.SemaphoreType.DMA] * (2 * NBUF),
    )


_sc_hop = _make_sc_hop(D_FEAT)


def _sc_prep_body(src_hbm, dst_hbm, aug_hbm, zeros_hbm, outD_hbm, outB_hbm,
                  si_v, di_v, rows_v, accD_sh, accB_sh, *sems):
    gsems = sems[:NBUF]
    ssems = sems[NBUF:]
    cid = lax.axis_index("c")
    sid = lax.axis_index("s")
    wid = sid * NC + cid
    strip = pl.ds(sid * ROWS_PER_TILE, ROWS_PER_TILE)

    pltpu.sync_copy(zeros_hbm.at[strip], accD_sh.at[strip])
    pltpu.sync_copy(zeros_hbm.at[strip], accB_sh.at[strip])
    plsc.subcore_barrier()

    for h in range(2):
        pltpu.sync_copy(src_hbm.at[pl.ds(wid * NK + h * NKH, NKH)], si_v)
        pltpu.sync_copy(dst_hbm.at[pl.ds(wid * NK + h * NKH, NKH)], di_v)

        def pair(j, carry):
            gds = []
            for b in range(NBUF):
                gds.append(pltpu.async_copy(
                    aug_hbm.at[di_v.at[j * NBUF + b]], rows_v.at[b],
                    gsems[b]))
            sds = []
            for b in range(NBUF):
                gds[b].wait()
                sds.append(pltpu.async_copy(
                    rows_v.at[b], accD_sh.at[si_v.at[j * NBUF + b]],
                    ssems[b], add=True))
            for b in range(NBUF):
                sds[b].wait()
                pltpu.sync_copy(rows_v.at[b],
                                accB_sh.at[di_v.at[j * NBUF + b]], add=True)
            return carry

        lax.fori_loop(0, NKH // NBUF, pair, 0)
    plsc.subcore_barrier()
    pltpu.sync_copy(accD_sh.at[strip], outD_hbm.at[cid, strip])
    pltpu.sync_copy(accB_sh.at[strip], outB_hbm.at[cid, strip])


_sc_prep = pl.kernel(
    _sc_prep_body,
    mesh=_MESH,
    out_type=(jax.ShapeDtypeStruct((NC, NPAD, 16), _f32),
              jax.ShapeDtypeStruct((NC, NPAD, 16), _f32)),
    compiler_params=pltpu.CompilerParams(use_tc_tiling_on_sc=False),
    scratch_types=[
        pltpu.VMEM((NKH, CH), _i32),
        pltpu.VMEM((NKH, CH), _i32),
        pltpu.VMEM((NBUF, CH, 16), _f32),
        pltpu.VMEM_SHARED((NPAD, 16), _f32),
        pltpu.VMEM_SHARED((NPAD, 16), _f32),
    ] + [pltpu.SemaphoreType.DMA] * (2 * NBUF),
)


def _bn_mm_gate_body(x_ref, g_ref, be_ref, w_ref, wg_ref, bg_ref,
                     xl_ref, gate_ref):
    x = x_ref[...]
    mu = jnp.mean(x, axis=0, keepdims=True)
    xc = x - mu
    var = jnp.mean(xc * xc, axis=0, keepdims=True)
    xn = xc * lax.rsqrt(var + 1e-5) * g_ref[...] + be_ref[...]
    xl_ref[0:N_NODES, :] = lax.dot_general(
        xn, w_ref[...], (((1,), (1,)), ((), ())),
        preferred_element_type=_f32)
    xl_ref[N_NODES:NPAD, :] = jnp.zeros((NPAD - N_NODES, D_FEAT), _f32)
    z = jnp.sum(x * wg_ref[...], axis=1, keepdims=True) + bg_ref[...]
    gate_ref[...] = 1.0 / (1.0 + jnp.exp(-z))


_tc_bn_mm_gate = pl.pallas_call(
    _bn_mm_gate_body,
    out_shape=(jax.ShapeDtypeStruct((NPAD, D_FEAT), _f32),
               jax.ShapeDtypeStruct((N_NODES, 1), _f32)),
)


def _safe_inv(d):
    return jnp.where(d == 0, 0.0, 1.0 / jnp.where(d == 0, 1.0, d))


def _merge_he_body(p_ref, accB_ref, he_ref):
    s = p_ref[0] + p_ref[1]
    b = accB_ref[0, :, 1:2] + accB_ref[1, :, 1:2]
    he_ref[...] = s * _safe_inv(b)


_tc_merge_he = pl.pallas_call(
    _merge_he_body,
    out_shape=jax.ShapeDtypeStruct((NPAD, D_FEAT), _f32),
)


def _update_body(q_ref, accD_ref, b_ref, gate_ref, x_ref, out_ref):
    s = q_ref[0, 0:N_NODES, :] + q_ref[1, 0:N_NODES, :]
    d = accD_ref[0, 0:N_NODES, 0:1] + accD_ref[1, 0:N_NODES, 0:1]
    h = jnp.maximum(s * _safe_inv(d) + b_ref[...], 0.0)
    out_ref[...] = x_ref[...] + h * gate_ref[...]


_tc_update = pl.pallas_call(
    _update_body,
    out_shape=jax.ShapeDtypeStruct((N_NODES, D_FEAT), _f32),
)


def _update_final_body(q_ref, accD_ref, b_ref, gate_ref, x_ref, x0_ref,
                       out_ref):
    s = q_ref[0, 0:N_NODES, :] + q_ref[1, 0:N_NODES, :]
    d = accD_ref[0, 0:N_NODES, 0:1] + accD_ref[1, 0:N_NODES, 0:1]
    h = jnp.maximum(s * _safe_inv(d) + b_ref[...], 0.0)
    xn = x_ref[...] + h * gate_ref[...]
    out_ref[...] = 2.0 * xn + x0_ref[...]


_tc_update_final = pl.pallas_call(
    _update_final_body,
    out_shape=jax.ShapeDtypeStruct((N_NODES, D_FEAT), _f32),
)


def kernel(X, H, hyperedge_weight,
           gamma0, beta0, W0, b0, wg0, bg0,
           gamma1, beta1, W1, b1, wg1, bg1,
           gamma2, beta2, W2, b2, wg2, bg2):
    npad_inc = N_INC_PAD - N_INC
    src = jnp.concatenate(
        [H[0].astype(_i32), jnp.zeros((npad_inc,), _i32)]).reshape(-1, CH)
    dst = jnp.concatenate(
        [H[1].astype(_i32),
         jnp.full((npad_inc,), TRASH_ROW, _i32)]).reshape(-1, CH)
    hw = hyperedge_weight.astype(_f32)
    aug = jnp.zeros((NPAD, 16), _f32)
    aug = aug.at[:N_HEDGES, 0].set(hw)
    aug = aug.at[:N_HEDGES, 1].set(1.0)
    zeros16 = jnp.zeros((NPAD, 16), _f32)
    zeros128 = jnp.zeros((NPAD, D_FEAT), _f32)

    accD, accB = _sc_prep(src, dst, aug, zeros16)

    params = [
        (gamma0, beta0, W0, b0, wg0, bg0),
        (gamma1, beta1, W1, b1, wg1, bg1),
        (gamma2, beta2, W2, b2, wg2, bg2),
    ]
    x0 = X
    x = X
    for layer, (g, be, W, b, wg, bg) in enumerate(params):
        xl, gate = _tc_bn_mm_gate(x, g.reshape(1, -1), be.reshape(1, -1),
                                  W, wg, bg.reshape(1, 1))
        p = _sc_hop(src, dst, xl, zeros128)
        he = _tc_merge_he(p, accB)
        q = _sc_hop(dst, src, he, zeros128)
        if layer < 2:
            x = _tc_update(q, accD, b.reshape(1, -1), gate, x)
        else:
            x = _tc_update_final(q, accD, b.reshape(1, -1), gate, x, x0)
    return x


# speedup vs baseline: 3.0146x; 3.0146x over previous
import functools

import jax
import jax.numpy as jnp
from jax import lax
from jax.experimental import pallas as pl
from jax.experimental.pallas import tpu as pltpu
from jax.experimental.pallas import tpu_sc as plsc

N_NODES = 10000
N_INC = 320000
D_FEAT = 128
N_HEDGES = 10000

NC = 2
NS = 16
NW = NC * NS
CH = 128
NK = 80
NKH = NK // 2
N_INC_PAD = NW * NK * CH
NPAD = 10112
TRASH_ROW = 10050
ROWS_PER_TILE = NPAD // NS
NBUF = 2

_f32 = jnp.float32
_i32 = jnp.int32

_MESH = plsc.VectorSubcoreMesh(
    core_axis_name="c", subcore_axis_name="s", num_cores=NC, num_subcores=NS)


def _sc_hop_body(gidx_hbm, widx_hbm, table_hbm, zeros_hbm, out_hbm,
                 gi_v, wi_v, rows_v, acc_sh, *sems):
    gsems = sems[:NBUF]
    ssems = sems[NBUF:]
    cid = lax.axis_index("c")
    sid = lax.axis_index("s")
    wid = sid * NC + cid
    strip = pl.ds(sid * ROWS_PER_TILE, ROWS_PER_TILE)

    pltpu.sync_copy(zeros_hbm.at[strip], acc_sh.at[strip])
    plsc.subcore_barrier()

    for h in range(2):
        pltpu.sync_copy(gidx_hbm.at[pl.ds(wid * NK + h * NKH, NKH)], gi_v)
        pltpu.sync_copy(widx_hbm.at[pl.ds(wid * NK + h * NKH, NKH)], wi_v)

        def pair(j, carry):
            gds = []
            for b in range(NBUF):
                gds.append(pltpu.async_copy(
                    table_hbm.at[gi_v.at[j * NBUF + b]], rows_v.at[b],
                    gsems[b]))
            sds = []
            for b in range(NBUF):
                gds[b].wait()
                sds.append(pltpu.async_copy(
                    rows_v.at[b], acc_sh.at[wi_v.at[j * NBUF + b]], ssems[b],
                    add=True))
            for b in range(NBUF):
                sds[b].wait()
            return carry

        lax.fori_loop(0, NKH // NBUF, pair, 0)
    plsc.subcore_barrier()
    pltpu.sync_copy(acc_sh.at[strip], out_hbm.at[cid, strip])


def _make_sc_hop(feat):
    return pl.kernel(
        _sc_hop_body,
        mesh=_MESH,
        out_type=jax.ShapeDtypeStruct((NC, NPAD, feat), _f32),
        compiler_params=pltpu.CompilerParams(use_tc_tiling_on_sc=False),
        scratch_types=[
            pltpu.VMEM((NKH, CH), _i32),
            pltpu.VMEM((NKH, CH), _i32),
            pltpu.VMEM((NBUF, CH, feat), _f32),
            pltpu.VMEM_SHARED((NPAD, feat), _f32),
        ] + [pltpu.SemaphoreType.DMA] * (2 * NBUF),
    )


_sc_hop = _make_sc_hop(D_FEAT)


def _sc_prep_body(src_hbm, dst_hbm, aug_hbm, zeros_hbm, outD_hbm, outB_hbm,
                  si_v, di_v, rows_v, accD_sh, accB_sh, *sems):
    gsems = sems[:NBUF]
    ssems = sems[NBUF:]
    cid = lax.axis_index("c")
    sid = lax.axis_index("s")
    wid = sid * NC + cid
    strip = pl.ds(sid * ROWS_PER_TILE, ROWS_PER_TILE)

    pltpu.sync_copy(zeros_hbm.at[strip], accD_sh.at[strip])
    pltpu.sync_copy(zeros_hbm.at[strip], accB_sh.at[strip])
    plsc.subcore_barrier()

    for h in range(2):
        pltpu.sync_copy(src_hbm.at[pl.ds(wid * NK + h * NKH, NKH)], si_v)
        pltpu.sync_copy(dst_hbm.at[pl.ds(wid * NK + h * NKH, NKH)], di_v)

        def pair(j, carry):
            gds = []
            for b in range(NBUF):
                gds.append(pltpu.async_copy(
                    aug_hbm.at[di_v.at[j * NBUF + b]], rows_v.at[b],
                    gsems[b]))
            sds = []
            for b in range(NBUF):
                gds[b].wait()
                sds.append(pltpu.async_copy(
                    rows_v.at[b], accD_sh.at[si_v.at[j * NBUF + b]],
                    ssems[b], add=True))
            for b in range(NBUF):
                sds[b].wait()
                pltpu.sync_copy(rows_v.at[b],
                                accB_sh.at[di_v.at[j * NBUF + b]], add=True)
            return carry

        lax.fori_loop(0, NKH // NBUF, pair, 0)
    plsc.subcore_barrier()
    pltpu.sync_copy(accD_sh.at[strip], outD_hbm.at[cid, strip])
    pltpu.sync_copy(accB_sh.at[strip], outB_hbm.at[cid, strip])


_sc_prep = pl.kernel(
    _sc_prep_body,
    mesh=_MESH,
    out_type=(jax.ShapeDtypeStruct((NC, NPAD, 16), _f32),
              jax.ShapeDtypeStruct((NC, NPAD, 16), _f32)),
    compiler_params=pltpu.CompilerParams(use_tc_tiling_on_sc=False),
    scratch_types=[
        pltpu.VMEM((NKH, CH), _i32),
        pltpu.VMEM((NKH, CH), _i32),
        pltpu.VMEM((NBUF, CH, 16), _f32),
        pltpu.VMEM_SHARED((NPAD, 16), _f32),
        pltpu.VMEM_SHARED((NPAD, 16), _f32),
    ] + [pltpu.SemaphoreType.DMA] * (2 * NBUF),
)


def _bn_mm_gate_body(x_ref, g_ref, be_ref, w_ref, wg_ref, bg_ref,
                     xl_ref, gate_ref):
    x = x_ref[...]
    mu = jnp.mean(x, axis=0, keepdims=True)
    xc = x - mu
    var = jnp.mean(xc * xc, axis=0, keepdims=True)
    xn = xc * lax.rsqrt(var + 1e-5) * g_ref[...] + be_ref[...]
    xl_ref[0:N_NODES, :] = lax.dot_general(
        xn, w_ref[...], (((1,), (1,)), ((), ())),
        preferred_element_type=_f32)
    xl_ref[N_NODES:NPAD, :] = jnp.zeros((NPAD - N_NODES, D_FEAT), _f32)
    z = jnp.sum(x * wg_ref[...], axis=1, keepdims=True) + bg_ref[...]
    gate_ref[...] = 1.0 / (1.0 + jnp.exp(-z))


_tc_bn_mm_gate = pl.pallas_call(
    _bn_mm_gate_body,
    out_shape=(jax.ShapeDtypeStruct((NPAD, D_FEAT), _f32),
               jax.ShapeDtypeStruct((N_NODES, 1), _f32)),
)


def _safe_inv(d):
    return jnp.where(d == 0, 0.0, 1.0 / jnp.where(d == 0, 1.0, d))


def _merge_he_body(p_ref, accB_ref, he_ref):
    s = p_ref[0] + p_ref[1]
    b = accB_ref[0, :, 1:2] + accB_ref[1, :, 1:2]
    he_ref[...] = s * _safe_inv(b)


_tc_merge_he = pl.pallas_call(
    _merge_he_body,
    out_shape=jax.ShapeDtypeStruct((NPAD, D_FEAT), _f32),
)


def _update_body(q_ref, accD_ref, b_ref, gate_ref, x_ref, out_ref):
    s = q_ref[0, 0:N_NODES, :] + q_ref[1, 0:N_NODES, :]
    d = accD_ref[0, 0:N_NODES, 0:1] + accD_ref[1, 0:N_NODES, 0:1]
    h = jnp.maximum(s * _safe_inv(d) + b_ref[...], 0.0)
    out_ref[...] = x_ref[...] + h * gate_ref[...]


_tc_update = pl.pallas_call(
    _update_body,
    out_shape=jax.ShapeDtypeStruct((N_NODES, D_FEAT), _f32),
)


def _update_final_body(q_ref, accD_ref, b_ref, gate_ref, x_ref, x0_ref,
                       out_ref):
    s = q_ref[0, 0:N_NODES, :] + q_ref[1, 0:N_NODES, :]
    d = accD_ref[0, 0:N_NODES, 0:1] + accD_ref[1, 0:N_NODES, 0:1]
    h = jnp.maximum(s * _safe_inv(d) + b_ref[...], 0.0)
    xn = x_ref[...] + h * gate_ref[...]
    out_ref[...] = 2.0 * xn + x0_ref[...]


_tc_update_final = pl.pallas_call(
    _update_final_body,
    out_shape=jax.ShapeDtypeStruct((N_NODES, D_FEAT), _f32),
)


def kernel(X, H, hyperedge_weight,
           gamma0, beta0, W0, b0, wg0, bg0,
           gamma1, beta1, W1, b1, wg1, bg1,
           gamma2, beta2, W2, b2, wg2, bg2):
    npad_inc = N_INC_PAD - N_INC
    pad_idx = N_NODES + jnp.arange(npad_inc, dtype=_i32) % (NPAD - N_NODES)
    src = jnp.concatenate([H[0].astype(_i32), pad_idx]).reshape(-1, CH)
    dst = jnp.concatenate([H[1].astype(_i32), pad_idx]).reshape(-1, CH)
    hw = hyperedge_weight.astype(_f32)
    aug = jnp.zeros((NPAD, 16), _f32)
    aug = aug.at[:N_HEDGES, 0].set(hw)
    aug = aug.at[:N_HEDGES, 1].set(1.0)
    zeros16 = jnp.zeros((NPAD, 16), _f32)
    zeros128 = jnp.zeros((NPAD, D_FEAT), _f32)

    accD, accB = _sc_prep(src, dst, aug, zeros16)

    params = [
        (gamma0, beta0, W0, b0, wg0, bg0),
        (gamma1, beta1, W1, b1, wg1, bg1),
        (gamma2, beta2, W2, b2, wg2, bg2),
    ]
    x0 = X
    x = X
    for layer, (g, be, W, b, wg, bg) in enumerate(params):
        xl, gate = _tc_bn_mm_gate(x, g.reshape(1, -1), be.reshape(1, -1),
                                  W, wg, bg.reshape(1, 1))
        p = _sc_hop(src, dst, xl, zeros128)
        he = _tc_merge_he(p, accB)
        q = _sc_hop(dst, src, he, zeros128)
        if layer < 2:
            x = _tc_update(q, accD, b.reshape(1, -1), gate, x)
        else:
            x = _tc_update_final(q, accD, b.reshape(1, -1), gate, x, x0)
    return x


# speedup vs baseline: 3.7179x; 1.2333x over previous
import functools

import jax
import jax.numpy as jnp
from jax import lax
from jax.experimental import pallas as pl
from jax.experimental.pallas import tpu as pltpu
from jax.experimental.pallas import tpu_sc as plsc

N_NODES = 10000
N_INC = 320000
D_FEAT = 128
N_HEDGES = 10000

NC = 2
NS = 16
NW = NC * NS
CH = 128
NK = 80
NKH = NK // 2
N_INC_PAD = NW * NK * CH
NPAD = 10112
TRASH_ROW = 10050
ROWS_PER_TILE = NPAD // NS
NBUF = 2

_f32 = jnp.float32
_i32 = jnp.int32

_MESH = plsc.VectorSubcoreMesh(
    core_axis_name="c", subcore_axis_name="s", num_cores=NC, num_subcores=NS)


def _sc_hop_body(gidx_hbm, widx_hbm, table_hbm, zeros_hbm, padidx_hbm,
                 out_hbm, gi_v, wi_v, rows_v, acc_sh, gsA, gsB, ssA, ssB):
    cid = lax.axis_index("c")
    sid = lax.axis_index("s")
    wid = sid * NC + cid
    strip = pl.ds(sid * ROWS_PER_TILE, ROWS_PER_TILE)

    pltpu.sync_copy(zeros_hbm.at[strip], acc_sh.at[strip])
    plsc.subcore_barrier()

    for h in range(2):
        pltpu.sync_copy(gidx_hbm.at[pl.ds(wid * NK + h * NKH, NKH)],
                        gi_v.at[pl.ds(0, NKH)])
        pltpu.sync_copy(widx_hbm.at[pl.ds(wid * NK + h * NKH, NKH)],
                        wi_v.at[pl.ds(0, NKH)])
        pltpu.sync_copy(padidx_hbm, gi_v.at[pl.ds(NKH, 1)])
        pltpu.sync_copy(padidx_hbm, wi_v.at[pl.ds(NKH, 1)])

        pltpu.async_copy(table_hbm.at[gi_v.at[0]], rows_v.at[0], gsA)
        pltpu.async_copy(rows_v.at[1], acc_sh.at[wi_v.at[NKH]], ssB,
                         add=True)

        def pair(j, carry):
            c0 = j * 2
            pltpu.make_async_copy(rows_v.at[1], acc_sh.at[wi_v.at[NKH]],
                                  ssB).wait()
            pltpu.async_copy(table_hbm.at[gi_v.at[c0 + 1]], rows_v.at[1],
                             gsB)
            pltpu.make_async_copy(table_hbm.at[gi_v.at[NKH]], rows_v.at[0],
                                  gsA).wait()
            pltpu.async_copy(rows_v.at[0], acc_sh.at[wi_v.at[c0]], ssA,
                             add=True)
            pltpu.make_async_copy(rows_v.at[0], acc_sh.at[wi_v.at[NKH]],
                                  ssA).wait()
            pltpu.async_copy(table_hbm.at[gi_v.at[c0 + 2]], rows_v.at[0],
                             gsA)
            pltpu.make_async_copy(table_hbm.at[gi_v.at[NKH]], rows_v.at[1],
                                  gsB).wait()
            pltpu.async_copy(rows_v.at[1], acc_sh.at[wi_v.at[c0 + 1]], ssB,
                             add=True)
            return carry

        lax.fori_loop(0, NKH // 2, pair, 0)
        pltpu.make_async_copy(table_hbm.at[gi_v.at[NKH]], rows_v.at[0],
                              gsA).wait()
        pltpu.make_async_copy(rows_v.at[1], acc_sh.at[wi_v.at[NKH]],
                              ssB).wait()
    plsc.subcore_barrier()
    pltpu.sync_copy(acc_sh.at[strip], out_hbm.at[cid, strip])


def _make_sc_hop(feat):
    return pl.kernel(
        _sc_hop_body,
        mesh=_MESH,
        out_type=jax.ShapeDtypeStruct((NC, NPAD, feat), _f32),
        compiler_params=pltpu.CompilerParams(use_tc_tiling_on_sc=False),
        scratch_types=[
            pltpu.VMEM((NKH + 1, CH), _i32),
            pltpu.VMEM((NKH + 1, CH), _i32),
            pltpu.VMEM((NBUF, CH, feat), _f32),
            pltpu.VMEM_SHARED((NPAD, feat), _f32),
        ] + [pltpu.SemaphoreType.DMA] * 4,
    )


_sc_hop = _make_sc_hop(D_FEAT)


def _sc_prep_body(src_hbm, dst_hbm, aug_hbm, zeros_hbm, outD_hbm, outB_hbm,
                  si_v, di_v, rows_v, accD_sh, accB_sh, *sems):
    gsems = sems[:NBUF]
    ssems = sems[NBUF:]
    cid = lax.axis_index("c")
    sid = lax.axis_index("s")
    wid = sid * NC + cid
    strip = pl.ds(sid * ROWS_PER_TILE, ROWS_PER_TILE)

    pltpu.sync_copy(zeros_hbm.at[strip], accD_sh.at[strip])
    pltpu.sync_copy(zeros_hbm.at[strip], accB_sh.at[strip])
    plsc.subcore_barrier()

    for h in range(2):
        pltpu.sync_copy(src_hbm.at[pl.ds(wid * NK + h * NKH, NKH)], si_v)
        pltpu.sync_copy(dst_hbm.at[pl.ds(wid * NK + h * NKH, NKH)], di_v)

        def pair(j, carry):
            gds = []
            for b in range(NBUF):
                gds.append(pltpu.async_copy(
                    aug_hbm.at[di_v.at[j * NBUF + b]], rows_v.at[b],
                    gsems[b]))
            sds = []
            for b in range(NBUF):
                gds[b].wait()
                sds.append(pltpu.async_copy(
                    rows_v.at[b], accD_sh.at[si_v.at[j * NBUF + b]],
                    ssems[b], add=True))
            for b in range(NBUF):
                sds[b].wait()
                pltpu.sync_copy(rows_v.at[b],
                                accB_sh.at[di_v.at[j * NBUF + b]], add=True)
            return carry

        lax.fori_loop(0, NKH // NBUF, pair, 0)
    plsc.subcore_barrier()
    pltpu.sync_copy(accD_sh.at[strip], outD_hbm.at[cid, strip])
    pltpu.sync_copy(accB_sh.at[strip], outB_hbm.at[cid, strip])


_sc_prep = pl.kernel(
    _sc_prep_body,
    mesh=_MESH,
    out_type=(jax.ShapeDtypeStruct((NC, NPAD, 16), _f32),
              jax.ShapeDtypeStruct((NC, NPAD, 16), _f32)),
    compiler_params=pltpu.CompilerParams(use_tc_tiling_on_sc=False),
    scratch_types=[
        pltpu.VMEM((NKH, CH), _i32),
        pltpu.VMEM((NKH, CH), _i32),
        pltpu.VMEM((NBUF, CH, 16), _f32),
        pltpu.VMEM_SHARED((NPAD, 16), _f32),
        pltpu.VMEM_SHARED((NPAD, 16), _f32),
    ] + [pltpu.SemaphoreType.DMA] * (2 * NBUF),
)


def _bn_mm_gate_body(x_ref, g_ref, be_ref, w_ref, wg_ref, bg_ref,
                     xl_ref, gate_ref):
    x = x_ref[...]
    mu = jnp.mean(x, axis=0, keepdims=True)
    xc = x - mu
    var = jnp.mean(xc * xc, axis=0, keepdims=True)
    xn = xc * lax.rsqrt(var + 1e-5) * g_ref[...] + be_ref[...]
    xl_ref[0:N_NODES, :] = lax.dot_general(
        xn, w_ref[...], (((1,), (1,)), ((), ())),
        preferred_element_type=_f32)
    xl_ref[N_NODES:NPAD, :] = jnp.zeros((NPAD - N_NODES, D_FEAT), _f32)
    z = jnp.sum(x * wg_ref[...], axis=1, keepdims=True) + bg_ref[...]
    gate_ref[...] = 1.0 / (1.0 + jnp.exp(-z))


_tc_bn_mm_gate = pl.pallas_call(
    _bn_mm_gate_body,
    out_shape=(jax.ShapeDtypeStruct((NPAD, D_FEAT), _f32),
               jax.ShapeDtypeStruct((N_NODES, 1), _f32)),
)


def _safe_inv(d):
    return jnp.where(d == 0, 0.0, 1.0 / jnp.where(d == 0, 1.0, d))


def _merge_he_body(p_ref, accB_ref, he_ref):
    s = p_ref[0] + p_ref[1]
    b = accB_ref[0, :, 1:2] + accB_ref[1, :, 1:2]
    he_ref[...] = s * _safe_inv(b)


_tc_merge_he = pl.pallas_call(
    _merge_he_body,
    out_shape=jax.ShapeDtypeStruct((NPAD, D_FEAT), _f32),
)


def _update_body(q_ref, accD_ref, b_ref, gate_ref, x_ref, out_ref):
    s = q_ref[0, 0:N_NODES, :] + q_ref[1, 0:N_NODES, :]
    d = accD_ref[0, 0:N_NODES, 0:1] + accD_ref[1, 0:N_NODES, 0:1]
    h = jnp.maximum(s * _safe_inv(d) + b_ref[...], 0.0)
    out_ref[...] = x_ref[...] + h * gate_ref[...]


_tc_update = pl.pallas_call(
    _update_body,
    out_shape=jax.ShapeDtypeStruct((N_NODES, D_FEAT), _f32),
)


def _update_final_body(q_ref, accD_ref, b_ref, gate_ref, x_ref, x0_ref,
                       out_ref):
    s = q_ref[0, 0:N_NODES, :] + q_ref[1, 0:N_NODES, :]
    d = accD_ref[0, 0:N_NODES, 0:1] + accD_ref[1, 0:N_NODES, 0:1]
    h = jnp.maximum(s * _safe_inv(d) + b_ref[...], 0.0)
    xn = x_ref[...] + h * gate_ref[...]
    out_ref[...] = 2.0 * xn + x0_ref[...]


_tc_update_final = pl.pallas_call(
    _update_final_body,
    out_shape=jax.ShapeDtypeStruct((N_NODES, D_FEAT), _f32),
)


def kernel(X, H, hyperedge_weight,
           gamma0, beta0, W0, b0, wg0, bg0,
           gamma1, beta1, W1, b1, wg1, bg1,
           gamma2, beta2, W2, b2, wg2, bg2):
    npad_inc = N_INC_PAD - N_INC
    pad_idx = N_NODES + jnp.arange(npad_inc, dtype=_i32) % (NPAD - N_NODES)
    src = jnp.concatenate([H[0].astype(_i32), pad_idx]).reshape(-1, CH)
    dst = jnp.concatenate([H[1].astype(_i32), pad_idx]).reshape(-1, CH)
    padidx = (N_NODES
              + jnp.arange(CH, dtype=_i32) % (NPAD - N_NODES)).reshape(1, CH)
    hw = hyperedge_weight.astype(_f32)
    aug = jnp.zeros((NPAD, 16), _f32)
    aug = aug.at[:N_HEDGES, 0].set(hw)
    aug = aug.at[:N_HEDGES, 1].set(1.0)
    zeros16 = jnp.zeros((NPAD, 16), _f32)
    zeros128 = jnp.zeros((NPAD, D_FEAT), _f32)

    accD, accB = _sc_prep(src, dst, aug, zeros16)

    params = [
        (gamma0, beta0, W0, b0, wg0, bg0),
        (gamma1, beta1, W1, b1, wg1, bg1),
        (gamma2, beta2, W2, b2, wg2, bg2),
    ]
    x0 = X
    x = X
    for layer, (g, be, W, b, wg, bg) in enumerate(params):
        xl, gate = _tc_bn_mm_gate(x, g.reshape(1, -1), be.reshape(1, -1),
                                  W, wg, bg.reshape(1, 1))
        p = _sc_hop(src, dst, xl, zeros128, padidx)
        he = _tc_merge_he(p, accB)
        q = _sc_hop(dst, src, he, zeros128, padidx)
        if layer < 2:
            x = _tc_update(q, accD, b.reshape(1, -1), gate, x)
        else:
            x = _tc_update_final(q, accD, b.reshape(1, -1), gate, x, x0)
    return x


# speedup vs baseline: 3.8159x; 1.0263x over previous
import functools

import jax
import jax.numpy as jnp
from jax import lax
from jax.experimental import pallas as pl
from jax.experimental.pallas import tpu as pltpu
from jax.experimental.pallas import tpu_sc as plsc

N_NODES = 10000
N_INC = 320000
D_FEAT = 128
N_HEDGES = 10000

NC = 2
NS = 16
NW = NC * NS
CH = 128
NK = 80
NKH = NK // 2
N_INC_PAD = NW * NK * CH
NPAD = 10112
TRASH_ROW = 10050
ROWS_PER_TILE = NPAD // NS
NBUF = 2

_f32 = jnp.float32
_i32 = jnp.int32

_MESH = plsc.VectorSubcoreMesh(
    core_axis_name="c", subcore_axis_name="s", num_cores=NC, num_subcores=NS)


def _sc_hop_body(gidx_hbm, widx_hbm, table_hbm, zeros_hbm, padidx_hbm,
                 out_hbm, gi_v, wi_v, rows_v, acc_sh, gsA, gsB, ssA, ssB):
    cid = lax.axis_index("c")
    sid = lax.axis_index("s")
    wid = sid * NC + cid
    strip = pl.ds(sid * ROWS_PER_TILE, ROWS_PER_TILE)

    zd = pltpu.async_copy(zeros_hbm.at[strip], acc_sh.at[strip], ssA)
    g0 = pltpu.async_copy(gidx_hbm.at[pl.ds(wid * NK, NKH)],
                          gi_v.at[pl.ds(0, NKH)], gsA)
    w0 = pltpu.async_copy(widx_hbm.at[pl.ds(wid * NK, NKH)],
                          wi_v.at[pl.ds(0, NKH)], gsB)
    pltpu.sync_copy(padidx_hbm, gi_v.at[pl.ds(NKH, 1)])
    pltpu.sync_copy(padidx_hbm, wi_v.at[pl.ds(NKH, 1)])
    zd.wait()
    g0.wait()
    w0.wait()
    plsc.subcore_barrier()

    for h in range(2):
        if h == 1:
            pltpu.sync_copy(gidx_hbm.at[pl.ds(wid * NK + NKH, NKH)],
                            gi_v.at[pl.ds(0, NKH)])
            pltpu.sync_copy(widx_hbm.at[pl.ds(wid * NK + NKH, NKH)],
                            wi_v.at[pl.ds(0, NKH)])

        pltpu.async_copy(table_hbm.at[gi_v.at[0]], rows_v.at[0], gsA)
        pltpu.async_copy(rows_v.at[1], acc_sh.at[wi_v.at[NKH]], ssB,
                         add=True)

        def pair(j, carry):
            c0 = j * 2
            pltpu.make_async_copy(rows_v.at[1], acc_sh.at[wi_v.at[NKH]],
                                  ssB).wait()
            pltpu.async_copy(table_hbm.at[gi_v.at[c0 + 1]], rows_v.at[1],
                             gsB)
            pltpu.make_async_copy(table_hbm.at[gi_v.at[NKH]], rows_v.at[0],
                                  gsA).wait()
            pltpu.async_copy(rows_v.at[0], acc_sh.at[wi_v.at[c0]], ssA,
                             add=True)
            pltpu.make_async_copy(rows_v.at[0], acc_sh.at[wi_v.at[NKH]],
                                  ssA).wait()
            pltpu.async_copy(table_hbm.at[gi_v.at[c0 + 2]], rows_v.at[0],
                             gsA)
            pltpu.make_async_copy(table_hbm.at[gi_v.at[NKH]], rows_v.at[1],
                                  gsB).wait()
            pltpu.async_copy(rows_v.at[1], acc_sh.at[wi_v.at[c0 + 1]], ssB,
                             add=True)
            return carry

        lax.fori_loop(0, NKH // 2, pair, 0)
        pltpu.make_async_copy(table_hbm.at[gi_v.at[NKH]], rows_v.at[0],
                              gsA).wait()
        pltpu.make_async_copy(rows_v.at[1], acc_sh.at[wi_v.at[NKH]],
                              ssB).wait()
    plsc.subcore_barrier()
    pltpu.sync_copy(acc_sh.at[strip], out_hbm.at[cid, strip])


def _make_sc_hop(feat):
    return pl.kernel(
        _sc_hop_body,
        mesh=_MESH,
        out_type=jax.ShapeDtypeStruct((NC, NPAD, feat), _f32),
        compiler_params=pltpu.CompilerParams(use_tc_tiling_on_sc=False),
        scratch_types=[
            pltpu.VMEM((NKH + 1, CH), _i32),
            pltpu.VMEM((NKH + 1, CH), _i32),
            pltpu.VMEM((NBUF, CH, feat), _f32),
            pltpu.VMEM_SHARED((NPAD, feat), _f32),
        ] + [pltpu.SemaphoreType.DMA] * 4,
    )


_sc_hop = _make_sc_hop(D_FEAT)


def _sc_prep_body(src_hbm, dst_hbm, aug_hbm, zeros_hbm, padidx_hbm,
                  outD_hbm, outB_hbm,
                  si_v, di_v, rows_v, accD_sh, accB_sh, gsA, gsB, ssA, ssB):
    cid = lax.axis_index("c")
    sid = lax.axis_index("s")
    wid = sid * NC + cid
    strip = pl.ds(sid * ROWS_PER_TILE, ROWS_PER_TILE)

    zd = pltpu.async_copy(zeros_hbm.at[strip], accD_sh.at[strip], ssA)
    zb = pltpu.async_copy(zeros_hbm.at[strip], accB_sh.at[strip], ssB)
    g0 = pltpu.async_copy(src_hbm.at[pl.ds(wid * NK, NKH)],
                          si_v.at[pl.ds(0, NKH)], gsA)
    w0 = pltpu.async_copy(dst_hbm.at[pl.ds(wid * NK, NKH)],
                          di_v.at[pl.ds(0, NKH)], gsB)
    pltpu.sync_copy(padidx_hbm, si_v.at[pl.ds(NKH, 1)])
    pltpu.sync_copy(padidx_hbm, di_v.at[pl.ds(NKH, 1)])
    zd.wait()
    zb.wait()
    g0.wait()
    w0.wait()
    plsc.subcore_barrier()

    for h in range(2):
        if h == 1:
            pltpu.sync_copy(src_hbm.at[pl.ds(wid * NK + NKH, NKH)],
                            si_v.at[pl.ds(0, NKH)])
            pltpu.sync_copy(dst_hbm.at[pl.ds(wid * NK + NKH, NKH)],
                            di_v.at[pl.ds(0, NKH)])

        pltpu.async_copy(aug_hbm.at[di_v.at[0]], rows_v.at[0], gsA)
        pltpu.async_copy(rows_v.at[1], accD_sh.at[si_v.at[NKH]], ssB,
                         add=True)
        pltpu.async_copy(rows_v.at[1], accB_sh.at[di_v.at[NKH]], ssB,
                         add=True)

        def pair(j, carry):
            c0 = j * 2
            pltpu.make_async_copy(rows_v.at[1], accD_sh.at[si_v.at[NKH]],
                                  ssB).wait()
            pltpu.make_async_copy(rows_v.at[1], accD_sh.at[si_v.at[NKH]],
                                  ssB).wait()
            pltpu.async_copy(aug_hbm.at[di_v.at[c0 + 1]], rows_v.at[1], gsB)
            pltpu.make_async_copy(aug_hbm.at[di_v.at[NKH]], rows_v.at[0],
                                  gsA).wait()
            pltpu.async_copy(rows_v.at[0], accD_sh.at[si_v.at[c0]], ssA,
                             add=True)
            pltpu.async_copy(rows_v.at[0], accB_sh.at[di_v.at[c0]], ssA,
                             add=True)
            pltpu.make_async_copy(rows_v.at[0], accD_sh.at[si_v.at[NKH]],
                                  ssA).wait()
            pltpu.make_async_copy(rows_v.at[0], accD_sh.at[si_v.at[NKH]],
                                  ssA).wait()
            pltpu.async_copy(aug_hbm.at[di_v.at[c0 + 2]], rows_v.at[0], gsA)
            pltpu.make_async_copy(aug_hbm.at[di_v.at[NKH]], rows_v.at[1],
                                  gsB).wait()
            pltpu.async_copy(rows_v.at[1], accD_sh.at[si_v.at[c0 + 1]], ssB,
                             add=True)
            pltpu.async_copy(rows_v.at[1], accB_sh.at[di_v.at[c0 + 1]], ssB,
                             add=True)
            return carry

        lax.fori_loop(0, NKH // 2, pair, 0)
        pltpu.make_async_copy(aug_hbm.at[di_v.at[NKH]], rows_v.at[0],
                              gsA).wait()
        pltpu.make_async_copy(rows_v.at[1], accD_sh.at[si_v.at[NKH]],
                              ssB).wait()
        pltpu.make_async_copy(rows_v.at[1], accD_sh.at[si_v.at[NKH]],
                              ssB).wait()
    plsc.subcore_barrier()
    pltpu.sync_copy(accD_sh.at[strip], outD_hbm.at[cid, strip])
    pltpu.sync_copy(accB_sh.at[strip], outB_hbm.at[cid, strip])


_sc_prep = pl.kernel(
    _sc_prep_body,
    mesh=_MESH,
    out_type=(jax.ShapeDtypeStruct((NC, NPAD, 16), _f32),
              jax.ShapeDtypeStruct((NC, NPAD, 16), _f32)),
    compiler_params=pltpu.CompilerParams(use_tc_tiling_on_sc=False),
    scratch_types=[
        pltpu.VMEM((NKH + 1, CH), _i32),
        pltpu.VMEM((NKH + 1, CH), _i32),
        pltpu.VMEM((NBUF, CH, 16), _f32),
        pltpu.VMEM_SHARED((NPAD, 16), _f32),
        pltpu.VMEM_SHARED((NPAD, 16), _f32),
    ] + [pltpu.SemaphoreType.DMA] * 4,
)


def _bn_mm_gate_body(x_ref, g_ref, be_ref, w_ref, wg_ref, bg_ref,
                     xl_ref, gate_ref):
    x = x_ref[...]
    mu = jnp.mean(x, axis=0, keepdims=True)
    xc = x - mu
    var = jnp.mean(xc * xc, axis=0, keepdims=True)
    xn = xc * lax.rsqrt(var + 1e-5) * g_ref[...] + be_ref[...]
    xl_ref[0:N_NODES, :] = lax.dot_general(
        xn, w_ref[...], (((1,), (1,)), ((), ())),
        preferred_element_type=_f32)
    xl_ref[N_NODES:NPAD, :] = jnp.zeros((NPAD - N_NODES, D_FEAT), _f32)
    z = jnp.sum(x * wg_ref[...], axis=1, keepdims=True) + bg_ref[...]
    gate_ref[...] = 1.0 / (1.0 + jnp.exp(-z))


_tc_bn_mm_gate = pl.pallas_call(
    _bn_mm_gate_body,
    out_shape=(jax.ShapeDtypeStruct((NPAD, D_FEAT), _f32),
               jax.ShapeDtypeStruct((N_NODES, 1), _f32)),
)


def _safe_inv(d):
    return jnp.where(d == 0, 0.0, 1.0 / jnp.where(d == 0, 1.0, d))


def _merge_he_body(p_ref, accB_ref, he_ref):
    s = p_ref[0] + p_ref[1]
    b = accB_ref[0, :, 1:2] + accB_ref[1, :, 1:2]
    he_ref[...] = s * _safe_inv(b)


_tc_merge_he = pl.pallas_call(
    _merge_he_body,
    out_shape=jax.ShapeDtypeStruct((NPAD, D_FEAT), _f32),
)


def _update_body(q_ref, accD_ref, b_ref, gate_ref, x_ref, out_ref):
    s = q_ref[0, 0:N_NODES, :] + q_ref[1, 0:N_NODES, :]
    d = accD_ref[0, 0:N_NODES, 0:1] + accD_ref[1, 0:N_NODES, 0:1]
    h = jnp.maximum(s * _safe_inv(d) + b_ref[...], 0.0)
    out_ref[...] = x_ref[...] + h * gate_ref[...]


_tc_update = pl.pallas_call(
    _update_body,
    out_shape=jax.ShapeDtypeStruct((N_NODES, D_FEAT), _f32),
)


def _update_final_body(q_ref, accD_ref, b_ref, gate_ref, x_ref, x0_ref,
                       out_ref):
    s = q_ref[0, 0:N_NODES, :] + q_ref[1, 0:N_NODES, :]
    d = accD_ref[0, 0:N_NODES, 0:1] + accD_ref[1, 0:N_NODES, 0:1]
    h = jnp.maximum(s * _safe_inv(d) + b_ref[...], 0.0)
    xn = x_ref[...] + h * gate_ref[...]
    out_ref[...] = 2.0 * xn + x0_ref[...]


_tc_update_final = pl.pallas_call(
    _update_final_body,
    out_shape=jax.ShapeDtypeStruct((N_NODES, D_FEAT), _f32),
)


def kernel(X, H, hyperedge_weight,
           gamma0, beta0, W0, b0, wg0, bg0,
           gamma1, beta1, W1, b1, wg1, bg1,
           gamma2, beta2, W2, b2, wg2, bg2):
    npad_inc = N_INC_PAD - N_INC
    pad_idx = N_NODES + jnp.arange(npad_inc, dtype=_i32) % (NPAD - N_NODES)
    src = jnp.concatenate([H[0].astype(_i32), pad_idx]).reshape(-1, CH)
    dst = jnp.concatenate([H[1].astype(_i32), pad_idx]).reshape(-1, CH)
    padidx = (N_NODES
              + jnp.arange(CH, dtype=_i32) % (NPAD - N_NODES)).reshape(1, CH)
    hw = hyperedge_weight.astype(_f32)
    aug = jnp.zeros((NPAD, 16), _f32)
    aug = aug.at[:N_HEDGES, 0].set(hw)
    aug = aug.at[:N_HEDGES, 1].set(1.0)
    zeros16 = jnp.zeros((NPAD, 16), _f32)
    zeros128 = jnp.zeros((NPAD, D_FEAT), _f32)

    accD, accB = _sc_prep(src, dst, aug, zeros16, padidx)

    params = [
        (gamma0, beta0, W0, b0, wg0, bg0),
        (gamma1, beta1, W1, b1, wg1, bg1),
        (gamma2, beta2, W2, b2, wg2, bg2),
    ]
    x0 = X
    x = X
    for layer, (g, be, W, b, wg, bg) in enumerate(params):
        xl, gate = _tc_bn_mm_gate(x, g.reshape(1, -1), be.reshape(1, -1),
                                  W, wg, bg.reshape(1, 1))
        p = _sc_hop(src, dst, xl, zeros128, padidx)
        he = _tc_merge_he(p, accB)
        q = _sc_hop(dst, src, he, zeros128, padidx)
        if layer < 2:
            x = _tc_update(q, accD, b.reshape(1, -1), gate, x)
        else:
            x = _tc_update_final(q, accD, b.reshape(1, -1), gate, x, x0)
    return x


# speedup vs baseline: 3.8760x; 1.0158x over previous
import functools

import jax
import jax.numpy as jnp
from jax import lax
from jax.experimental import pallas as pl
from jax.experimental.pallas import tpu as pltpu
from jax.experimental.pallas import tpu_sc as plsc

N_NODES = 10000
N_INC = 320000
D_FEAT = 128
N_HEDGES = 10000

NC = 2
NS = 16
NW = NC * NS
CH = 128
NK = 80
NKH = NK // 2
N_INC_PAD = NW * NK * CH
NPAD = 10112
TRASH_ROW = 10050
ROWS_PER_TILE = NPAD // NS
NBUF = 2

_f32 = jnp.float32
_i32 = jnp.int32

_MESH = plsc.VectorSubcoreMesh(
    core_axis_name="c", subcore_axis_name="s", num_cores=NC, num_subcores=NS)


def _sc_hop_body(gidx_hbm, widx_hbm, table_hbm, zeros_hbm, padidx_hbm,
                 out_hbm, gi_v, wi_v, rows_v, acc_sh, gsA, gsB, ssA, ssB):
    cid = lax.axis_index("c")
    sid = lax.axis_index("s")
    wid = sid * NC + cid
    strip = pl.ds(sid * ROWS_PER_TILE, ROWS_PER_TILE)

    zd = pltpu.async_copy(zeros_hbm.at[strip], acc_sh.at[strip], ssA)
    g0 = pltpu.async_copy(gidx_hbm.at[pl.ds(wid * NK, NKH)],
                          gi_v.at[pl.ds(0, NKH)], gsA)
    w0 = pltpu.async_copy(widx_hbm.at[pl.ds(wid * NK, NKH)],
                          wi_v.at[pl.ds(0, NKH)], gsB)
    pltpu.sync_copy(padidx_hbm, gi_v.at[pl.ds(NKH, 1)])
    pltpu.sync_copy(padidx_hbm, wi_v.at[pl.ds(NKH, 1)])
    zd.wait()
    g0.wait()
    w0.wait()
    plsc.subcore_barrier()

    for h in range(2):
        if h == 1:
            pltpu.sync_copy(gidx_hbm.at[pl.ds(wid * NK + NKH, NKH)],
                            gi_v.at[pl.ds(0, NKH)])
            pltpu.sync_copy(widx_hbm.at[pl.ds(wid * NK + NKH, NKH)],
                            wi_v.at[pl.ds(0, NKH)])

        pltpu.async_copy(table_hbm.at[gi_v.at[0]], rows_v.at[0], gsA)
        pltpu.async_copy(rows_v.at[1], acc_sh.at[wi_v.at[NKH]], ssB,
                         add=True)

        def pair(j, carry):
            c0 = j * 2
            pltpu.make_async_copy(rows_v.at[1], acc_sh.at[wi_v.at[NKH]],
                                  ssB).wait()
            pltpu.async_copy(table_hbm.at[gi_v.at[c0 + 1]], rows_v.at[1],
                             gsB)
            pltpu.make_async_copy(table_hbm.at[gi_v.at[NKH]], rows_v.at[0],
                                  gsA).wait()
            pltpu.async_copy(rows_v.at[0], acc_sh.at[wi_v.at[c0]], ssA,
                             add=True)
            pltpu.make_async_copy(rows_v.at[0], acc_sh.at[wi_v.at[NKH]],
                                  ssA).wait()
            pltpu.async_copy(table_hbm.at[gi_v.at[c0 + 2]], rows_v.at[0],
                             gsA)
            pltpu.make_async_copy(table_hbm.at[gi_v.at[NKH]], rows_v.at[1],
                                  gsB).wait()
            pltpu.async_copy(rows_v.at[1], acc_sh.at[wi_v.at[c0 + 1]], ssB,
                             add=True)
            return carry

        lax.fori_loop(0, NKH // 2, pair, 0)
        pltpu.make_async_copy(table_hbm.at[gi_v.at[NKH]], rows_v.at[0],
                              gsA).wait()
        pltpu.make_async_copy(rows_v.at[1], acc_sh.at[wi_v.at[NKH]],
                              ssB).wait()
    plsc.subcore_barrier()
    pltpu.sync_copy(acc_sh.at[strip], out_hbm.at[cid, strip])


def _make_sc_hop(feat):
    return pl.kernel(
        _sc_hop_body,
        mesh=_MESH,
        out_type=jax.ShapeDtypeStruct((NC, NPAD, feat), _f32),
        compiler_params=pltpu.CompilerParams(use_tc_tiling_on_sc=False),
        scratch_types=[
            pltpu.VMEM((NKH + 1, CH), _i32),
            pltpu.VMEM((NKH + 1, CH), _i32),
            pltpu.VMEM((NBUF, CH, feat), _f32),
            pltpu.VMEM_SHARED((NPAD, feat), _f32),
        ] + [pltpu.SemaphoreType.DMA] * 4,
    )


_sc_hop = _make_sc_hop(D_FEAT)


def _sc_prep_body(src_hbm, dst_hbm, aug_hbm, zeros_hbm, padidx_hbm,
                  outD_hbm, outB_hbm,
                  si_v, di_v, rows_v, accD_sh, accB_sh, gsA, gsB, ssA, ssB):
    cid = lax.axis_index("c")
    sid = lax.axis_index("s")
    wid = sid * NC + cid
    strip = pl.ds(sid * ROWS_PER_TILE, ROWS_PER_TILE)

    zd = pltpu.async_copy(zeros_hbm.at[strip], accD_sh.at[strip], ssA)
    zb = pltpu.async_copy(zeros_hbm.at[strip], accB_sh.at[strip], ssB)
    g0 = pltpu.async_copy(src_hbm.at[pl.ds(wid * NK, NKH)],
                          si_v.at[pl.ds(0, NKH)], gsA)
    w0 = pltpu.async_copy(dst_hbm.at[pl.ds(wid * NK, NKH)],
                          di_v.at[pl.ds(0, NKH)], gsB)
    pltpu.sync_copy(padidx_hbm, si_v.at[pl.ds(NKH, 1)])
    pltpu.sync_copy(padidx_hbm, di_v.at[pl.ds(NKH, 1)])
    zd.wait()
    zb.wait()
    g0.wait()
    w0.wait()
    plsc.subcore_barrier()

    for h in range(2):
        if h == 1:
            pltpu.sync_copy(src_hbm.at[pl.ds(wid * NK + NKH, NKH)],
                            si_v.at[pl.ds(0, NKH)])
            pltpu.sync_copy(dst_hbm.at[pl.ds(wid * NK + NKH, NKH)],
                            di_v.at[pl.ds(0, NKH)])

        pltpu.async_copy(aug_hbm.at[di_v.at[0]], rows_v.at[0], gsA)
        pltpu.async_copy(rows_v.at[1], accD_sh.at[si_v.at[NKH]], ssB,
                         add=True)
        pltpu.async_copy(rows_v.at[1], accB_sh.at[di_v.at[NKH]], ssB,
                         add=True)

        def pair(j, carry):
            c0 = j * 2
            pltpu.make_async_copy(rows_v.at[1], accD_sh.at[si_v.at[NKH]],
                                  ssB).wait()
            pltpu.make_async_copy(rows_v.at[1], accD_sh.at[si_v.at[NKH]],
                                  ssB).wait()
            pltpu.async_copy(aug_hbm.at[di_v.at[c0 + 1]], rows_v.at[1], gsB)
            pltpu.make_async_copy(aug_hbm.at[di_v.at[NKH]], rows_v.at[0],
                                  gsA).wait()
            pltpu.async_copy(rows_v.at[0], accD_sh.at[si_v.at[c0]], ssA,
                             add=True)
            pltpu.async_copy(rows_v.at[0], accB_sh.at[di_v.at[c0]], ssA,
                             add=True)
            pltpu.make_async_copy(rows_v.at[0], accD_sh.at[si_v.at[NKH]],
                                  ssA).wait()
            pltpu.make_async_copy(rows_v.at[0], accD_sh.at[si_v.at[NKH]],
                                  ssA).wait()
            pltpu.async_copy(aug_hbm.at[di_v.at[c0 + 2]], rows_v.at[0], gsA)
            pltpu.make_async_copy(aug_hbm.at[di_v.at[NKH]], rows_v.at[1],
                                  gsB).wait()
            pltpu.async_copy(rows_v.at[1], accD_sh.at[si_v.at[c0 + 1]], ssB,
                             add=True)
            pltpu.async_copy(rows_v.at[1], accB_sh.at[di_v.at[c0 + 1]], ssB,
                             add=True)
            return carry

        lax.fori_loop(0, NKH // 2, pair, 0)
        pltpu.make_async_copy(aug_hbm.at[di_v.at[NKH]], rows_v.at[0],
                              gsA).wait()
        pltpu.make_async_copy(rows_v.at[1], accD_sh.at[si_v.at[NKH]],
                              ssB).wait()
        pltpu.make_async_copy(rows_v.at[1], accD_sh.at[si_v.at[NKH]],
                              ssB).wait()
    plsc.subcore_barrier()
    pltpu.sync_copy(accD_sh.at[strip], outD_hbm.at[cid, strip])
    pltpu.sync_copy(accB_sh.at[strip], outB_hbm.at[cid, strip])


_sc_prep = pl.kernel(
    _sc_prep_body,
    mesh=_MESH,
    out_type=(jax.ShapeDtypeStruct((NC, NPAD, 16), _f32),
              jax.ShapeDtypeStruct((NC, NPAD, 16), _f32)),
    compiler_params=pltpu.CompilerParams(use_tc_tiling_on_sc=False),
    scratch_types=[
        pltpu.VMEM((NKH + 1, CH), _i32),
        pltpu.VMEM((NKH + 1, CH), _i32),
        pltpu.VMEM((NBUF, CH, 16), _f32),
        pltpu.VMEM_SHARED((NPAD, 16), _f32),
        pltpu.VMEM_SHARED((NPAD, 16), _f32),
    ] + [pltpu.SemaphoreType.DMA] * 4,
)


def _bn_mm_gate_body(x_ref, g_ref, be_ref, w_ref, wg_ref, bg_ref,
                     xl_ref, gate_ref):
    x = x_ref[...]
    mu = jnp.mean(x, axis=0, keepdims=True)
    xc = x - mu
    var = jnp.mean(xc * xc, axis=0, keepdims=True)
    xn = xc * lax.rsqrt(var + 1e-5) * g_ref[...] + be_ref[...]
    xl_ref[0:N_NODES, :] = lax.dot_general(
        xn, w_ref[...], (((1,), (1,)), ((), ())),
        preferred_element_type=_f32)
    xl_ref[N_NODES:NPAD, :] = jnp.zeros((NPAD - N_NODES, D_FEAT), _f32)
    z = jnp.sum(x * wg_ref[...], axis=1, keepdims=True) + bg_ref[...]
    gate_ref[...] = 1.0 / (1.0 + jnp.exp(-z))


_tc_bn_mm_gate = pl.pallas_call(
    _bn_mm_gate_body,
    out_shape=(jax.ShapeDtypeStruct((NPAD, D_FEAT), _f32),
               jax.ShapeDtypeStruct((N_NODES, 1), _f32)),
)


def _safe_inv(d):
    return jnp.where(d == 0, 0.0, 1.0 / jnp.where(d == 0, 1.0, d))


def _merge_he_body(p_ref, accB_ref, he_ref):
    s = p_ref[0] + p_ref[1]
    b = accB_ref[0, :, 1:2] + accB_ref[1, :, 1:2]
    he_ref[...] = s * _safe_inv(b)


_tc_merge_he = pl.pallas_call(
    _merge_he_body,
    out_shape=jax.ShapeDtypeStruct((NPAD, D_FEAT), _f32),
)


def _update_bn_body(q_ref, accD_ref, b_ref, gate_ref, x_ref,
                    g_ref, be_ref, w_ref, wg_ref, bg_ref,
                    xn_ref, xl_ref, gate2_ref):
    s = q_ref[0, 0:N_NODES, :] + q_ref[1, 0:N_NODES, :]
    d = accD_ref[0, 0:N_NODES, 0:1] + accD_ref[1, 0:N_NODES, 0:1]
    h = jnp.maximum(s * _safe_inv(d) + b_ref[...], 0.0)
    x = x_ref[...] + h * gate_ref[...]
    xn_ref[...] = x
    mu = jnp.mean(x, axis=0, keepdims=True)
    xc = x - mu
    var = jnp.mean(xc * xc, axis=0, keepdims=True)
    xb = xc * lax.rsqrt(var + 1e-5) * g_ref[...] + be_ref[...]
    xl_ref[0:N_NODES, :] = lax.dot_general(
        xb, w_ref[...], (((1,), (1,)), ((), ())),
        preferred_element_type=_f32)
    xl_ref[N_NODES:NPAD, :] = jnp.zeros((NPAD - N_NODES, D_FEAT), _f32)
    z = jnp.sum(x * wg_ref[...], axis=1, keepdims=True) + bg_ref[...]
    gate2_ref[...] = 1.0 / (1.0 + jnp.exp(-z))


_tc_update_bn = pl.pallas_call(
    _update_bn_body,
    out_shape=(jax.ShapeDtypeStruct((N_NODES, D_FEAT), _f32),
               jax.ShapeDtypeStruct((NPAD, D_FEAT), _f32),
               jax.ShapeDtypeStruct((N_NODES, 1), _f32)),
)


def _update_final_body(q_ref, accD_ref, b_ref, gate_ref, x_ref, x0_ref,
                       out_ref):
    s = q_ref[0, 0:N_NODES, :] + q_ref[1, 0:N_NODES, :]
    d = accD_ref[0, 0:N_NODES, 0:1] + accD_ref[1, 0:N_NODES, 0:1]
    h = jnp.maximum(s * _safe_inv(d) + b_ref[...], 0.0)
    xn = x_ref[...] + h * gate_ref[...]
    out_ref[...] = 2.0 * xn + x0_ref[...]


_tc_update_final = pl.pallas_call(
    _update_final_body,
    out_shape=jax.ShapeDtypeStruct((N_NODES, D_FEAT), _f32),
)


def kernel(X, H, hyperedge_weight,
           gamma0, beta0, W0, b0, wg0, bg0,
           gamma1, beta1, W1, b1, wg1, bg1,
           gamma2, beta2, W2, b2, wg2, bg2):
    npad_inc = N_INC_PAD - N_INC
    pad_idx = N_NODES + jnp.arange(npad_inc, dtype=_i32) % (NPAD - N_NODES)
    src = jnp.concatenate([H[0].astype(_i32), pad_idx]).reshape(-1, CH)
    dst = jnp.concatenate([H[1].astype(_i32), pad_idx]).reshape(-1, CH)
    padidx = (N_NODES
              + jnp.arange(CH, dtype=_i32) % (NPAD - N_NODES)).reshape(1, CH)
    hw = hyperedge_weight.astype(_f32)
    aug = jnp.zeros((NPAD, 16), _f32)
    aug = aug.at[:N_HEDGES, 0].set(hw)
    aug = aug.at[:N_HEDGES, 1].set(1.0)
    zeros16 = jnp.zeros((NPAD, 16), _f32)
    zeros128 = jnp.zeros((NPAD, D_FEAT), _f32)

    accD, accB = _sc_prep(src, dst, aug, zeros16, padidx)

    params = [
        (gamma0, beta0, W0, b0, wg0, bg0),
        (gamma1, beta1, W1, b1, wg1, bg1),
        (gamma2, beta2, W2, b2, wg2, bg2),
    ]
    x0 = X
    x = X
    xl, gate = _tc_bn_mm_gate(x, gamma0.reshape(1, -1), beta0.reshape(1, -1),
                              W0, wg0, bg0.reshape(1, 1))
    for layer, (g, be, W, b, wg, bg) in enumerate(params):
        p = _sc_hop(src, dst, xl, zeros128, padidx)
        he = _tc_merge_he(p, accB)
        q = _sc_hop(dst, src, he, zeros128, padidx)
        if layer < 2:
            g2, be2, W2n, _, wg2n, bg2n = params[layer + 1]
            x, xl, gate = _tc_update_bn(
                q, accD, b.reshape(1, -1), gate, x,
                g2.reshape(1, -1), be2.reshape(1, -1), W2n, wg2n,
                bg2n.reshape(1, 1))
        else:
            x = _tc_update_final(q, accD, b.reshape(1, -1), gate, x, x0)
    return x


# speedup vs baseline: 4.0194x; 1.0370x over previous
import functools

import jax
import jax.numpy as jnp
from jax import lax
from jax.experimental import pallas as pl
from jax.experimental.pallas import tpu as pltpu
from jax.experimental.pallas import tpu_sc as plsc

N_NODES = 10000
N_INC = 320000
D_FEAT = 128
N_HEDGES = 10000

NC = 2
NS = 16
NW = NC * NS
CH = 128
NK = 80
NKH = NK // 2
N_INC_PAD = NW * NK * CH
NPAD = 10112
TRASH_ROW = 10050
ROWS_PER_TILE = NPAD // NS
NBUF = 2

_f32 = jnp.float32
_i32 = jnp.int32

_MESH = plsc.VectorSubcoreMesh(
    core_axis_name="c", subcore_axis_name="s", num_cores=NC, num_subcores=NS)


def _sc_hop_body(gidx_hbm, widx_hbm, table_hbm, zeros_hbm, padidx_hbm,
                 out_hbm, gi_v, wi_v, rows_v, acc_sh, gsA, gsB, ssA, ssB):
    cid = lax.axis_index("c")
    sid = lax.axis_index("s")
    wid = sid * NC + cid
    strip = pl.ds(sid * ROWS_PER_TILE, ROWS_PER_TILE)

    zd = pltpu.async_copy(zeros_hbm.at[strip], acc_sh.at[strip], ssA)
    g0 = pltpu.async_copy(gidx_hbm.at[pl.ds(wid * NK, NKH)],
                          gi_v.at[pl.ds(0, NKH)], gsA)
    w0 = pltpu.async_copy(widx_hbm.at[pl.ds(wid * NK, NKH)],
                          wi_v.at[pl.ds(0, NKH)], gsB)
    pltpu.sync_copy(padidx_hbm, gi_v.at[pl.ds(NKH, 1)])
    pltpu.sync_copy(padidx_hbm, wi_v.at[pl.ds(NKH, 1)])
    zd.wait()
    g0.wait()
    w0.wait()
    plsc.subcore_barrier()

    for h in range(2):
        if h == 1:
            pltpu.sync_copy(gidx_hbm.at[pl.ds(wid * NK + NKH, NKH)],
                            gi_v.at[pl.ds(0, NKH)])
            pltpu.sync_copy(widx_hbm.at[pl.ds(wid * NK + NKH, NKH)],
                            wi_v.at[pl.ds(0, NKH)])

        pltpu.async_copy(table_hbm.at[gi_v.at[0]], rows_v.at[0], gsA)
        pltpu.async_copy(rows_v.at[1], acc_sh.at[wi_v.at[NKH]], ssB,
                         add=True)

        def pair(j, carry):
            c0 = j * 2
            pltpu.make_async_copy(rows_v.at[1], acc_sh.at[wi_v.at[NKH]],
                                  ssB).wait()
            pltpu.async_copy(table_hbm.at[gi_v.at[c0 + 1]], rows_v.at[1],
                             gsB)
            pltpu.make_async_copy(table_hbm.at[gi_v.at[NKH]], rows_v.at[0],
                                  gsA).wait()
            pltpu.async_copy(rows_v.at[0], acc_sh.at[wi_v.at[c0]], ssA,
                             add=True)
            pltpu.make_async_copy(rows_v.at[0], acc_sh.at[wi_v.at[NKH]],
                                  ssA).wait()
            pltpu.async_copy(table_hbm.at[gi_v.at[c0 + 2]], rows_v.at[0],
                             gsA)
            pltpu.make_async_copy(table_hbm.at[gi_v.at[NKH]], rows_v.at[1],
                                  gsB).wait()
            pltpu.async_copy(rows_v.at[1], acc_sh.at[wi_v.at[c0 + 1]], ssB,
                             add=True)
            return carry

        lax.fori_loop(0, NKH // 2, pair, 0)
        pltpu.make_async_copy(table_hbm.at[gi_v.at[NKH]], rows_v.at[0],
                              gsA).wait()
        pltpu.make_async_copy(rows_v.at[1], acc_sh.at[wi_v.at[NKH]],
                              ssB).wait()
    plsc.subcore_barrier()
    pltpu.sync_copy(acc_sh.at[strip], out_hbm.at[cid, strip])


def _make_sc_hop(feat):
    return pl.kernel(
        _sc_hop_body,
        mesh=_MESH,
        out_type=jax.ShapeDtypeStruct((NC, NPAD, feat), _f32),
        compiler_params=pltpu.CompilerParams(use_tc_tiling_on_sc=False),
        scratch_types=[
            pltpu.VMEM((NKH + 1, CH), _i32),
            pltpu.VMEM((NKH + 1, CH), _i32),
            pltpu.VMEM((NBUF, CH, feat), _f32),
            pltpu.VMEM_SHARED((NPAD, feat), _f32),
        ] + [pltpu.SemaphoreType.DMA] * 4,
    )


_sc_hop = _make_sc_hop(D_FEAT)


def _sc_prep_body(src_hbm, dst_hbm, aug_hbm, zeros_hbm, padidx_hbm,
                  outD_hbm, outB_hbm,
                  si_v, di_v, rows_v, accD_sh, accB_sh, gsA, gsB, ssA, ssB):
    cid = lax.axis_index("c")
    sid = lax.axis_index("s")
    wid = sid * NC + cid
    strip = pl.ds(sid * ROWS_PER_TILE, ROWS_PER_TILE)

    zd = pltpu.async_copy(zeros_hbm.at[strip], accD_sh.at[strip], ssA)
    zb = pltpu.async_copy(zeros_hbm.at[strip], accB_sh.at[strip], ssB)
    g0 = pltpu.async_copy(src_hbm.at[pl.ds(wid * NK, NKH)],
                          si_v.at[pl.ds(0, NKH)], gsA)
    w0 = pltpu.async_copy(dst_hbm.at[pl.ds(wid * NK, NKH)],
                          di_v.at[pl.ds(0, NKH)], gsB)
    pltpu.sync_copy(padidx_hbm, si_v.at[pl.ds(NKH, 1)])
    pltpu.sync_copy(padidx_hbm, di_v.at[pl.ds(NKH, 1)])
    zd.wait()
    zb.wait()
    g0.wait()
    w0.wait()
    plsc.subcore_barrier()

    for h in range(2):
        if h == 1:
            pltpu.sync_copy(src_hbm.at[pl.ds(wid * NK + NKH, NKH)],
                            si_v.at[pl.ds(0, NKH)])
            pltpu.sync_copy(dst_hbm.at[pl.ds(wid * NK + NKH, NKH)],
                            di_v.at[pl.ds(0, NKH)])

        pltpu.async_copy(aug_hbm.at[di_v.at[0]], rows_v.at[0], gsA)
        pltpu.async_copy(rows_v.at[1], accD_sh.at[si_v.at[NKH]], ssB,
                         add=True)
        pltpu.async_copy(rows_v.at[1], accB_sh.at[di_v.at[NKH]], ssB,
                         add=True)

        def pair(j, carry):
            c0 = j * 2
            pltpu.make_async_copy(rows_v.at[1], accD_sh.at[si_v.at[NKH]],
                                  ssB).wait()
            pltpu.make_async_copy(rows_v.at[1], accD_sh.at[si_v.at[NKH]],
                                  ssB).wait()
            pltpu.async_copy(aug_hbm.at[di_v.at[c0 + 1]], rows_v.at[1], gsB)
            pltpu.make_async_copy(aug_hbm.at[di_v.at[NKH]], rows_v.at[0],
                                  gsA).wait()
            pltpu.async_copy(rows_v.at[0], accD_sh.at[si_v.at[c0]], ssA,
                             add=True)
            pltpu.async_copy(rows_v.at[0], accB_sh.at[di_v.at[c0]], ssA,
                             add=True)
            pltpu.make_async_copy(rows_v.at[0], accD_sh.at[si_v.at[NKH]],
                                  ssA).wait()
            pltpu.make_async_copy(rows_v.at[0], accD_sh.at[si_v.at[NKH]],
                                  ssA).wait()
            pltpu.async_copy(aug_hbm.at[di_v.at[c0 + 2]], rows_v.at[0], gsA)
            pltpu.make_async_copy(aug_hbm.at[di_v.at[NKH]], rows_v.at[1],
                                  gsB).wait()
            pltpu.async_copy(rows_v.at[1], accD_sh.at[si_v.at[c0 + 1]], ssB,
                             add=True)
            pltpu.async_copy(rows_v.at[1], accB_sh.at[di_v.at[c0 + 1]], ssB,
                             add=True)
            return carry

        lax.fori_loop(0, NKH // 2, pair, 0)
        pltpu.make_async_copy(aug_hbm.at[di_v.at[NKH]], rows_v.at[0],
                              gsA).wait()
        pltpu.make_async_copy(rows_v.at[1], accD_sh.at[si_v.at[NKH]],
                              ssB).wait()
        pltpu.make_async_copy(rows_v.at[1], accD_sh.at[si_v.at[NKH]],
                              ssB).wait()
    plsc.subcore_barrier()
    pltpu.sync_copy(accD_sh.at[strip], outD_hbm.at[cid, strip])
    pltpu.sync_copy(accB_sh.at[strip], outB_hbm.at[cid, strip])


_sc_prep = pl.kernel(
    _sc_prep_body,
    mesh=_MESH,
    out_type=(jax.ShapeDtypeStruct((NC, NPAD, 16), _f32),
              jax.ShapeDtypeStruct((NC, NPAD, 16), _f32)),
    compiler_params=pltpu.CompilerParams(use_tc_tiling_on_sc=False),
    scratch_types=[
        pltpu.VMEM((NKH + 1, CH), _i32),
        pltpu.VMEM((NKH + 1, CH), _i32),
        pltpu.VMEM((NBUF, CH, 16), _f32),
        pltpu.VMEM_SHARED((NPAD, 16), _f32),
        pltpu.VMEM_SHARED((NPAD, 16), _f32),
    ] + [pltpu.SemaphoreType.DMA] * 4,
)


def _bn_mm_gate_body(x_ref, g_ref, be_ref, w_ref, wg_ref, bg_ref,
                     xl_ref, gate_ref):
    x = x_ref[...]
    mu = jnp.mean(x, axis=0, keepdims=True)
    xc = x - mu
    var = jnp.mean(xc * xc, axis=0, keepdims=True)
    xn = xc * lax.rsqrt(var + 1e-5) * g_ref[...] + be_ref[...]
    xl_ref[0:N_NODES, :] = lax.dot_general(
        xn, w_ref[...], (((1,), (1,)), ((), ())),
        preferred_element_type=_f32)
    xl_ref[N_NODES:NPAD, :] = jnp.zeros((NPAD - N_NODES, D_FEAT), _f32)
    z = jnp.sum(x * wg_ref[...], axis=1, keepdims=True) + bg_ref[...]
    gate_ref[...] = 1.0 / (1.0 + jnp.exp(-z))


_tc_bn_mm_gate = pl.pallas_call(
    _bn_mm_gate_body,
    out_shape=(jax.ShapeDtypeStruct((NPAD, D_FEAT), _f32),
               jax.ShapeDtypeStruct((N_NODES, 1), _f32)),
)


def _safe_inv(d):
    return jnp.where(d == 0, 0.0, 1.0 / jnp.where(d == 0, 1.0, d))


_N_REAL_CH = N_INC // CH
_N_PAD_CH = NW * NK - _N_REAL_CH


def _prep_inputs_body(h_ref, hw_ref, src_ref, dst_ref, aug_ref, padidx_ref):
    row = lax.broadcasted_iota(_i32, (_N_PAD_CH, CH), 0)
    lane = lax.broadcasted_iota(_i32, (_N_PAD_CH, CH), 1)
    padv = N_NODES + lax.rem(row * CH + lane, NPAD - N_NODES)
    src_ref[0:_N_REAL_CH, :] = h_ref[0]
    src_ref[_N_REAL_CH:, :] = padv
    dst_ref[0:_N_REAL_CH, :] = h_ref[1]
    dst_ref[_N_REAL_CH:, :] = padv
    padidx_ref[...] = N_NODES + lax.rem(
        lax.broadcasted_iota(_i32, (1, CH), 1), NPAD - N_NODES)
    hwp = jnp.concatenate(
        [hw_ref[...], jnp.zeros((NPAD - N_HEDGES, 1), _f32)], axis=0)
    aug_ref[...] = jnp.concatenate(
        [hwp, jnp.ones((NPAD, 1), _f32), jnp.zeros((NPAD, 14), _f32)],
        axis=1)


_tc_prep_inputs = pl.pallas_call(
    _prep_inputs_body,
    out_shape=(jax.ShapeDtypeStruct((NW * NK, CH), _i32),
               jax.ShapeDtypeStruct((NW * NK, CH), _i32),
               jax.ShapeDtypeStruct((NPAD, 16), _f32),
               jax.ShapeDtypeStruct((1, CH), _i32)),
)


def _merge_he_body(p_ref, accB_ref, he_ref):
    s = p_ref[0] + p_ref[1]
    b = accB_ref[0, :, 1:2] + accB_ref[1, :, 1:2]
    he_ref[...] = s * _safe_inv(b)


_tc_merge_he = pl.pallas_call(
    _merge_he_body,
    out_shape=jax.ShapeDtypeStruct((NPAD, D_FEAT), _f32),
)


def _update_bn_body(q_ref, accD_ref, b_ref, gate_ref, x_ref,
                    g_ref, be_ref, w_ref, wg_ref, bg_ref,
                    xn_ref, xl_ref, gate2_ref):
    s = q_ref[0, 0:N_NODES, :] + q_ref[1, 0:N_NODES, :]
    d = accD_ref[0, 0:N_NODES, 0:1] + accD_ref[1, 0:N_NODES, 0:1]
    h = jnp.maximum(s * _safe_inv(d) + b_ref[...], 0.0)
    x = x_ref[...] + h * gate_ref[...]
    xn_ref[...] = x
    mu = jnp.mean(x, axis=0, keepdims=True)
    xc = x - mu
    var = jnp.mean(xc * xc, axis=0, keepdims=True)
    xb = xc * lax.rsqrt(var + 1e-5) * g_ref[...] + be_ref[...]
    xl_ref[0:N_NODES, :] = lax.dot_general(
        xb, w_ref[...], (((1,), (1,)), ((), ())),
        preferred_element_type=_f32)
    xl_ref[N_NODES:NPAD, :] = jnp.zeros((NPAD - N_NODES, D_FEAT), _f32)
    z = jnp.sum(x * wg_ref[...], axis=1, keepdims=True) + bg_ref[...]
    gate2_ref[...] = 1.0 / (1.0 + jnp.exp(-z))


_tc_update_bn = pl.pallas_call(
    _update_bn_body,
    out_shape=(jax.ShapeDtypeStruct((N_NODES, D_FEAT), _f32),
               jax.ShapeDtypeStruct((NPAD, D_FEAT), _f32),
               jax.ShapeDtypeStruct((N_NODES, 1), _f32)),
)


def _update_final_body(q_ref, accD_ref, b_ref, gate_ref, x_ref, x0_ref,
                       out_ref):
    s = q_ref[0, 0:N_NODES, :] + q_ref[1, 0:N_NODES, :]
    d = accD_ref[0, 0:N_NODES, 0:1] + accD_ref[1, 0:N_NODES, 0:1]
    h = jnp.maximum(s * _safe_inv(d) + b_ref[...], 0.0)
    xn = x_ref[...] + h * gate_ref[...]
    out_ref[...] = 2.0 * xn + x0_ref[...]


_tc_update_final = pl.pallas_call(
    _update_final_body,
    out_shape=jax.ShapeDtypeStruct((N_NODES, D_FEAT), _f32),
)


def kernel(X, H, hyperedge_weight,
           gamma0, beta0, W0, b0, wg0, bg0,
           gamma1, beta1, W1, b1, wg1, bg1,
           gamma2, beta2, W2, b2, wg2, bg2):
    src, dst, aug, padidx = _tc_prep_inputs(
        H.astype(_i32).reshape(2, _N_REAL_CH, CH),
        hyperedge_weight.astype(_f32).reshape(N_HEDGES, 1))
    zeros16 = jnp.zeros((NPAD, 16), _f32)
    zeros128 = jnp.zeros((NPAD, D_FEAT), _f32)

    accD, accB = _sc_prep(src, dst, aug, zeros16, padidx)

    params = [
        (gamma0, beta0, W0, b0, wg0, bg0),
        (gamma1, beta1, W1, b1, wg1, bg1),
        (gamma2, beta2, W2, b2, wg2, bg2),
    ]
    x0 = X
    x = X
    xl, gate = _tc_bn_mm_gate(x, gamma0.reshape(1, -1), beta0.reshape(1, -1),
                              W0, wg0, bg0.reshape(1, 1))
    for layer, (g, be, W, b, wg, bg) in enumerate(params):
        p = _sc_hop(src, dst, xl, zeros128, padidx)
        he = _tc_merge_he(p, accB)
        q = _sc_hop(dst, src, he, zeros128, padidx)
        if layer < 2:
            g2, be2, W2n, _, wg2n, bg2n = params[layer + 1]
            x, xl, gate = _tc_update_bn(
                q, accD, b.reshape(1, -1), gate, x,
                g2.reshape(1, -1), be2.reshape(1, -1), W2n, wg2n,
                bg2n.reshape(1, 1))
        else:
            x = _tc_update_final(q, accD, b.reshape(1, -1), gate, x, x0)
    return x


# speedup vs baseline: 4.0270x; 1.0019x over previous
import functools

import jax
import jax.numpy as jnp
from jax import lax
from jax.experimental import pallas as pl
from jax.experimental.pallas import tpu as pltpu
from jax.experimental.pallas import tpu_sc as plsc

N_NODES = 10000
N_INC = 320000
D_FEAT = 128
N_HEDGES = 10000

NC = 2
NS = 16
NW = NC * NS
CH = 128
NK = 80
NKH = NK // 2
N_INC_PAD = NW * NK * CH
NPAD = 10112
TRASH_ROW = 10050
ROWS_PER_TILE = NPAD // NS
NBUF = 2

_f32 = jnp.float32
_i32 = jnp.int32

_MESH = plsc.VectorSubcoreMesh(
    core_axis_name="c", subcore_axis_name="s", num_cores=NC, num_subcores=NS)


def _sc_hop_body(gidx_hbm, widx_hbm, table_hbm, zeros_hbm, padidx_hbm,
                 out_hbm, gi_v, wi_v, rows_v, acc_sh, gsA, gsB, ssA, ssB):
    cid = lax.axis_index("c")
    sid = lax.axis_index("s")
    wid = sid * NC + cid
    strip = pl.ds(sid * ROWS_PER_TILE, ROWS_PER_TILE)

    zd = pltpu.async_copy(zeros_hbm.at[strip], acc_sh.at[strip], ssA)
    g0 = pltpu.async_copy(gidx_hbm.at[pl.ds(wid * NK, NKH)],
                          gi_v.at[pl.ds(0, NKH)], gsA)
    w0 = pltpu.async_copy(widx_hbm.at[pl.ds(wid * NK, NKH)],
                          wi_v.at[pl.ds(0, NKH)], gsB)
    pltpu.sync_copy(padidx_hbm, gi_v.at[pl.ds(NKH, 1)])
    pltpu.sync_copy(padidx_hbm, wi_v.at[pl.ds(NKH, 1)])
    zd.wait()
    g0.wait()
    w0.wait()
    plsc.subcore_barrier()

    for h in range(2):
        if h == 1:
            pltpu.sync_copy(gidx_hbm.at[pl.ds(wid * NK + NKH, NKH)],
                            gi_v.at[pl.ds(0, NKH)])
            pltpu.sync_copy(widx_hbm.at[pl.ds(wid * NK + NKH, NKH)],
                            wi_v.at[pl.ds(0, NKH)])

        pltpu.async_copy(table_hbm.at[gi_v.at[0]], rows_v.at[0], gsA)
        pltpu.async_copy(rows_v.at[1], acc_sh.at[wi_v.at[NKH]], ssB,
                         add=True)

        def pair(j, carry):
            c0 = j * 2
            pltpu.make_async_copy(rows_v.at[1], acc_sh.at[wi_v.at[NKH]],
                                  ssB).wait()
            pltpu.async_copy(table_hbm.at[gi_v.at[c0 + 1]], rows_v.at[1],
                             gsB)
            pltpu.make_async_copy(table_hbm.at[gi_v.at[NKH]], rows_v.at[0],
                                  gsA).wait()
            pltpu.async_copy(rows_v.at[0], acc_sh.at[wi_v.at[c0]], ssA,
                             add=True)
            pltpu.make_async_copy(rows_v.at[0], acc_sh.at[wi_v.at[NKH]],
                                  ssA).wait()
            pltpu.async_copy(table_hbm.at[gi_v.at[c0 + 2]], rows_v.at[0],
                             gsA)
            pltpu.make_async_copy(table_hbm.at[gi_v.at[NKH]], rows_v.at[1],
                                  gsB).wait()
            pltpu.async_copy(rows_v.at[1], acc_sh.at[wi_v.at[c0 + 1]], ssB,
                             add=True)
            return carry

        lax.fori_loop(0, NKH // 2, pair, 0)
        pltpu.make_async_copy(table_hbm.at[gi_v.at[NKH]], rows_v.at[0],
                              gsA).wait()
        pltpu.make_async_copy(rows_v.at[1], acc_sh.at[wi_v.at[NKH]],
                              ssB).wait()
    plsc.subcore_barrier()
    pltpu.sync_copy(acc_sh.at[strip], out_hbm.at[cid, strip])


def _make_sc_hop(feat):
    return pl.kernel(
        _sc_hop_body,
        mesh=_MESH,
        out_type=jax.ShapeDtypeStruct((NC, NPAD, feat), _f32),
        compiler_params=pltpu.CompilerParams(use_tc_tiling_on_sc=False),
        scratch_types=[
            pltpu.VMEM((NKH + 1, CH), _i32),
            pltpu.VMEM((NKH + 1, CH), _i32),
            pltpu.VMEM((NBUF, CH, feat), _f32),
            pltpu.VMEM_SHARED((NPAD, feat), _f32),
        ] + [pltpu.SemaphoreType.DMA] * 4,
    )


_sc_hop = _make_sc_hop(D_FEAT)


def _sc_prep_body(src_hbm, dst_hbm, aug_hbm, zeros_hbm, padidx_hbm,
                  outD_hbm, outB_hbm,
                  si_v, di_v, rows_v, accD_sh, accB_sh, gsA, gsB, ssA, ssB):
    cid = lax.axis_index("c")
    sid = lax.axis_index("s")
    wid = sid * NC + cid
    strip = pl.ds(sid * ROWS_PER_TILE, ROWS_PER_TILE)

    zd = pltpu.async_copy(zeros_hbm.at[strip], accD_sh.at[strip], ssA)
    zb = pltpu.async_copy(zeros_hbm.at[strip], accB_sh.at[strip], ssB)
    g0 = pltpu.async_copy(src_hbm.at[pl.ds(wid * NK, NKH)],
                          si_v.at[pl.ds(0, NKH)], gsA)
    w0 = pltpu.async_copy(dst_hbm.at[pl.ds(wid * NK, NKH)],
                          di_v.at[pl.ds(0, NKH)], gsB)
    pltpu.sync_copy(padidx_hbm, si_v.at[pl.ds(NKH, 1)])
    pltpu.sync_copy(padidx_hbm, di_v.at[pl.ds(NKH, 1)])
    zd.wait()
    zb.wait()
    g0.wait()
    w0.wait()
    plsc.subcore_barrier()

    for h in range(2):
        if h == 1:
            pltpu.sync_copy(src_hbm.at[pl.ds(wid * NK + NKH, NKH)],
                            si_v.at[pl.ds(0, NKH)])
            pltpu.sync_copy(dst_hbm.at[pl.ds(wid * NK + NKH, NKH)],
                            di_v.at[pl.ds(0, NKH)])

        pltpu.async_copy(aug_hbm.at[di_v.at[0]], rows_v.at[0], gsA)
        pltpu.async_copy(rows_v.at[1], accD_sh.at[si_v.at[NKH]], ssB,
                         add=True)
        pltpu.async_copy(rows_v.at[1], accB_sh.at[di_v.at[NKH]], ssB,
                         add=True)

        def pair(j, carry):
            c0 = j * 2
            pltpu.make_async_copy(rows_v.at[1], accD_sh.at[si_v.at[NKH]],
                                  ssB).wait()
            pltpu.make_async_copy(rows_v.at[1], accD_sh.at[si_v.at[NKH]],
                                  ssB).wait()
            pltpu.async_copy(aug_hbm.at[di_v.at[c0 + 1]], rows_v.at[1], gsB)
            pltpu.make_async_copy(aug_hbm.at[di_v.at[NKH]], rows_v.at[0],
                                  gsA).wait()
            pltpu.async_copy(rows_v.at[0], accD_sh.at[si_v.at[c0]], ssA,
                             add=True)
            pltpu.async_copy(rows_v.at[0], accB_sh.at[di_v.at[c0]], ssA,
                             add=True)
            pltpu.make_async_copy(rows_v.at[0], accD_sh.at[si_v.at[NKH]],
                                  ssA).wait()
            pltpu.make_async_copy(rows_v.at[0], accD_sh.at[si_v.at[NKH]],
                                  ssA).wait()
            pltpu.async_copy(aug_hbm.at[di_v.at[c0 + 2]], rows_v.at[0], gsA)
            pltpu.make_async_copy(aug_hbm.at[di_v.at[NKH]], rows_v.at[1],
                                  gsB).wait()
            pltpu.async_copy(rows_v.at[1], accD_sh.at[si_v.at[c0 + 1]], ssB,
                             add=True)
            pltpu.async_copy(rows_v.at[1], accB_sh.at[di_v.at[c0 + 1]], ssB,
                             add=True)
            return carry

        lax.fori_loop(0, NKH // 2, pair, 0)
        pltpu.make_async_copy(aug_hbm.at[di_v.at[NKH]], rows_v.at[0],
                              gsA).wait()
        pltpu.make_async_copy(rows_v.at[1], accD_sh.at[si_v.at[NKH]],
                              ssB).wait()
        pltpu.make_async_copy(rows_v.at[1], accD_sh.at[si_v.at[NKH]],
                              ssB).wait()
    plsc.subcore_barrier()
    pltpu.sync_copy(accD_sh.at[strip], outD_hbm.at[cid, strip])
    pltpu.sync_copy(accB_sh.at[strip], outB_hbm.at[cid, strip])


_sc_prep = pl.kernel(
    _sc_prep_body,
    mesh=_MESH,
    out_type=(jax.ShapeDtypeStruct((NC, NPAD, 8), _f32),
              jax.ShapeDtypeStruct((NC, NPAD, 8), _f32)),
    compiler_params=pltpu.CompilerParams(use_tc_tiling_on_sc=False),
    scratch_types=[
        pltpu.VMEM((NKH + 1, CH), _i32),
        pltpu.VMEM((NKH + 1, CH), _i32),
        pltpu.VMEM((NBUF, CH, 8), _f32),
        pltpu.VMEM_SHARED((NPAD, 8), _f32),
        pltpu.VMEM_SHARED((NPAD, 8), _f32),
    ] + [pltpu.SemaphoreType.DMA] * 4,
)


def _bn_mm_gate_body(x_ref, g_ref, be_ref, w_ref, wg_ref, bg_ref,
                     xl_ref, gate_ref):
    x = x_ref[...]
    mu = jnp.mean(x, axis=0, keepdims=True)
    xc = x - mu
    var = jnp.mean(xc * xc, axis=0, keepdims=True)
    xn = xc * lax.rsqrt(var + 1e-5) * g_ref[...] + be_ref[...]
    xl_ref[0:N_NODES, :] = lax.dot_general(
        xn, w_ref[...], (((1,), (1,)), ((), ())),
        preferred_element_type=_f32)
    xl_ref[N_NODES:NPAD, :] = jnp.zeros((NPAD - N_NODES, D_FEAT), _f32)
    z = jnp.sum(x * wg_ref[...], axis=1, keepdims=True) + bg_ref[...]
    gate_ref[...] = 1.0 / (1.0 + jnp.exp(-z))


_tc_bn_mm_gate = pl.pallas_call(
    _bn_mm_gate_body,
    out_shape=(jax.ShapeDtypeStruct((NPAD, D_FEAT), _f32),
               jax.ShapeDtypeStruct((N_NODES, 1), _f32)),
)


def _safe_inv(d):
    return jnp.where(d == 0, 0.0, 1.0 / jnp.where(d == 0, 1.0, d))


_N_REAL_CH = N_INC // CH
_N_PAD_CH = NW * NK - _N_REAL_CH


def _prep_inputs_body(h_ref, hw_ref, src_ref, dst_ref, aug_ref, padidx_ref):
    row = lax.broadcasted_iota(_i32, (_N_PAD_CH, CH), 0)
    lane = lax.broadcasted_iota(_i32, (_N_PAD_CH, CH), 1)
    padv = N_NODES + lax.rem(row * CH + lane, NPAD - N_NODES)
    src_ref[0:_N_REAL_CH, :] = h_ref[0]
    src_ref[_N_REAL_CH:, :] = padv
    dst_ref[0:_N_REAL_CH, :] = h_ref[1]
    dst_ref[_N_REAL_CH:, :] = padv
    padidx_ref[...] = N_NODES + lax.rem(
        lax.broadcasted_iota(_i32, (1, CH), 1), NPAD - N_NODES)
    hwp = jnp.concatenate(
        [hw_ref[...], jnp.zeros((NPAD - N_HEDGES, 1), _f32)], axis=0)
    aug_ref[...] = jnp.concatenate(
        [hwp, jnp.ones((NPAD, 1), _f32), jnp.zeros((NPAD, 6), _f32)],
        axis=1)


_tc_prep_inputs = pl.pallas_call(
    _prep_inputs_body,
    out_shape=(jax.ShapeDtypeStruct((NW * NK, CH), _i32),
               jax.ShapeDtypeStruct((NW * NK, CH), _i32),
               jax.ShapeDtypeStruct((NPAD, 8), _f32),
               jax.ShapeDtypeStruct((1, CH), _i32)),
)


def _merge_he_body(p_ref, accB_ref, he_ref):
    s = p_ref[0] + p_ref[1]
    b = accB_ref[0, :, 1:2] + accB_ref[1, :, 1:2]
    he_ref[...] = s * _safe_inv(b)


_tc_merge_he = pl.pallas_call(
    _merge_he_body,
    out_shape=jax.ShapeDtypeStruct((NPAD, D_FEAT), _f32),
)


def _update_bn_body(q_ref, accD_ref, b_ref, gate_ref, x_ref,
                    g_ref, be_ref, w_ref, wg_ref, bg_ref,
                    xn_ref, xl_ref, gate2_ref):
    s = q_ref[0, 0:N_NODES, :] + q_ref[1, 0:N_NODES, :]
    d = accD_ref[0, 0:N_NODES, 0:1] + accD_ref[1, 0:N_NODES, 0:1]
    h = jnp.maximum(s * _safe_inv(d) + b_ref[...], 0.0)
    x = x_ref[...] + h * gate_ref[...]
    xn_ref[...] = x
    mu = jnp.mean(x, axis=0, keepdims=True)
    xc = x - mu
    var = jnp.mean(xc * xc, axis=0, keepdims=True)
    xb = xc * lax.rsqrt(var + 1e-5) * g_ref[...] + be_ref[...]
    xl_ref[0:N_NODES, :] = lax.dot_general(
        xb, w_ref[...], (((1,), (1,)), ((), ())),
        preferred_element_type=_f32)
    xl_ref[N_NODES:NPAD, :] = jnp.zeros((NPAD - N_NODES, D_FEAT), _f32)
    z = jnp.sum(x * wg_ref[...], axis=1, keepdims=True) + bg_ref[...]
    gate2_ref[...] = 1.0 / (1.0 + jnp.exp(-z))


_tc_update_bn = pl.pallas_call(
    _update_bn_body,
    out_shape=(jax.ShapeDtypeStruct((N_NODES, D_FEAT), _f32),
               jax.ShapeDtypeStruct((NPAD, D_FEAT), _f32),
               jax.ShapeDtypeStruct((N_NODES, 1), _f32)),
)


def _update_final_body(q_ref, accD_ref, b_ref, gate_ref, x_ref, x0_ref,
                       out_ref):
    s = q_ref[0, 0:N_NODES, :] + q_ref[1, 0:N_NODES, :]
    d = accD_ref[0, 0:N_NODES, 0:1] + accD_ref[1, 0:N_NODES, 0:1]
    h = jnp.maximum(s * _safe_inv(d) + b_ref[...], 0.0)
    xn = x_ref[...] + h * gate_ref[...]
    out_ref[...] = 2.0 * xn + x0_ref[...]


_tc_update_final = pl.pallas_call(
    _update_final_body,
    out_shape=jax.ShapeDtypeStruct((N_NODES, D_FEAT), _f32),
)


def kernel(X, H, hyperedge_weight,
           gamma0, beta0, W0, b0, wg0, bg0,
           gamma1, beta1, W1, b1, wg1, bg1,
           gamma2, beta2, W2, b2, wg2, bg2):
    src, dst, aug, padidx = _tc_prep_inputs(
        H.astype(_i32).reshape(2, _N_REAL_CH, CH),
        hyperedge_weight.astype(_f32).reshape(N_HEDGES, 1))
    zeros16 = jnp.zeros((NPAD, 8), _f32)
    zeros128 = jnp.zeros((NPAD, D_FEAT), _f32)

    accD, accB = _sc_prep(src, dst, aug, zeros16, padidx)

    params = [
        (gamma0, beta0, W0, b0, wg0, bg0),
        (gamma1, beta1, W1, b1, wg1, bg1),
        (gamma2, beta2, W2, b2, wg2, bg2),
    ]
    x0 = X
    x = X
    xl, gate = _tc_bn_mm_gate(x, gamma0.reshape(1, -1), beta0.reshape(1, -1),
                              W0, wg0, bg0.reshape(1, 1))
    for layer, (g, be, W, b, wg, bg) in enumerate(params):
        p = _sc_hop(src, dst, xl, zeros128, padidx)
        he = _tc_merge_he(p, accB)
        q = _sc_hop(dst, src, he, zeros128, padidx)
        if layer < 2:
            g2, be2, W2n, _, wg2n, bg2n = params[layer + 1]
            x, xl, gate = _tc_update_bn(
                q, accD, b.reshape(1, -1), gate, x,
                g2.reshape(1, -1), be2.reshape(1, -1), W2n, wg2n,
                bg2n.reshape(1, 1))
        else:
            x = _tc_update_final(q, accD, b.reshape(1, -1), gate, x, x0)
    return x
